# baseline (speedup 1.0000x reference)
import functools

import jax
import jax.numpy as jnp
from jax import lax
from jax.experimental import pallas as pl
from jax.experimental.pallas import tpu as pltpu
from jax.experimental.pallas import tpu_sc as plsc

V = 10000
E = 320000
DN = 128
DE = 16
H = 50
HP = 64
HQ = 128
GF = 128
G = 128
K = 11

_MEANS = (-1.645, -1.080, -0.739, -0.468, -0.228, 0.0, 0.228, 0.468, 0.739, 1.080, 1.645)
_STDS = (0.283, 0.170, 0.134, 0.118, 0.114, 0.114, 0.114, 0.118, 0.134, 0.170, 0.283)

EBLK = 8000
NEB = E // EBLK
VBLK = 1000
NVB = V // VBLK


def _pad2(W, rows, cols):
    r, c = W.shape
    return jnp.pad(W, ((0, rows - r), (0, cols - c)))


def _pad_b(b, cols):
    return jnp.pad(b, (0, cols - b.shape[0])).reshape(1, cols)


_NC = 2
_NS = 16
_CH = 80
_EPW = E // (_NC * _NS)
_NCH = _EPW // _CH
_VPAD = 10240
_VPS = _VPAD // _NS
_VOS = V // _NS


def _sc_segment_sum(data, idx):
    mesh = plsc.VectorSubcoreMesh(core_axis_name="c", subcore_axis_name="s")

    @functools.partial(
        pl.kernel, mesh=mesh,
        out_type=jax.ShapeDtypeStruct((_NC, _VPAD, HQ), jnp.float32),
        scratch_types=[
            pltpu.VMEM_SHARED((_VPAD, HQ), jnp.float32),
            pltpu.VMEM((128, HQ), jnp.float32),
            pltpu.VMEM((_CH,), jnp.int32), pltpu.VMEM((_CH,), jnp.int32),
            pltpu.VMEM((_CH, HQ), jnp.float32), pltpu.VMEM((_CH, HQ), jnp.float32),
            pltpu.SemaphoreType.DMA, pltpu.SemaphoreType.DMA,
            pltpu.SemaphoreType.DMA, pltpu.SemaphoreType.DMA,
        ],
    )
    def k(data_hbm, idx_hbm, out_hbm, acc_sh, zbuf, idx0, idx1, rows0, rows1,
          si0, si1, sr0, sr1):
        c = lax.axis_index("c")
        s = lax.axis_index("s")
        z16 = jnp.zeros((16,), jnp.float32)

        def zrow(i, carry):
            for j in range(HQ // 16):
                zbuf[i, pl.ds(j * 16, 16)] = z16
            return carry

        lax.fori_loop(0, 128, zrow, 0)
        for m in range(_VPS // 128):
            pltpu.sync_copy(zbuf, acc_sh.at[pl.ds(s * _VPS + m * 128, 128)])
        plsc.subcore_barrier()

        base = (c * _NS + s) * _EPW
        idxb = (idx0, idx1)
        rowsb = (rows0, rows1)
        sib = (si0, si1)
        srb = (sr0, sr1)

        def start(kk, b):
            off = pl.multiple_of(base + kk * _CH, 8)
            pltpu.async_copy(idx_hbm.at[pl.ds(off, _CH)], idxb[b], sib[b])
            pltpu.async_copy(data_hbm.at[pl.ds(off, _CH)], rowsb[b], srb[b])

        def wait(b):
            pltpu.make_async_copy(idx_hbm.at[pl.ds(0, _CH)], idxb[b], sib[b]).wait()
            pltpu.make_async_copy(data_hbm.at[pl.ds(0, _CH)], rowsb[b], srb[b]).wait()

        def scat(b):
            pltpu.sync_copy(rowsb[b], acc_sh.at[idxb[b]], add=True)

        start(0, 0)
        start(1, 1)

        def body(k2, carry):
            for b in range(2):
                kk = k2 * 2 + b
                wait(b)
                scat(b)

                @pl.when(kk + 2 < _NCH)
                def _():
                    start(kk + 2, b)
            return carry

        lax.fori_loop(0, (_NCH - 1) // 2, body, 0)
        wait(0)
        scat(0)
        plsc.subcore_barrier()
        pltpu.sync_copy(acc_sh.at[pl.ds(s * _VPS, _VPS)],
                        out_hbm.at[c, pl.ds(s * _VPS, _VPS)])

    return k(data, idx)


def _sc_edge_gather(lr, src, dst):
    mesh = plsc.VectorSubcoreMesh(core_axis_name="c", subcore_axis_name="s")

    @functools.partial(
        pl.kernel, mesh=mesh,
        out_type=jax.ShapeDtypeStruct((E, HP), jnp.float32),
        scratch_types=[
            pltpu.VMEM((_CH,), jnp.int32), pltpu.VMEM((_CH,), jnp.int32),
            pltpu.VMEM((_CH,), jnp.int32), pltpu.VMEM((_CH,), jnp.int32),
            pltpu.VMEM((_CH, 2 * HP), jnp.float32), pltpu.VMEM((_CH, 2 * HP), jnp.float32),
            pltpu.VMEM((_CH, 2 * HP), jnp.float32), pltpu.VMEM((_CH, 2 * HP), jnp.float32),
            pltpu.VMEM((_CH, HP), jnp.float32), pltpu.VMEM((_CH, HP), jnp.float32),
            pltpu.SemaphoreType.DMA, pltpu.SemaphoreType.DMA,
            pltpu.SemaphoreType.DMA, pltpu.SemaphoreType.DMA,
            pltpu.SemaphoreType.DMA, pltpu.SemaphoreType.DMA,
            pltpu.SemaphoreType.DMA, pltpu.SemaphoreType.DMA,
        ],
    )
    def k(lr_hbm, src_hbm, dst_hbm, out_hbm, ixs0, ixs1, ixd0, ixd1, srow0, srow1,
          drow0, drow1, fsb0, fsb1, si0, si1, sgs0, sgs1, sgd0, sgd1, sw0, sw1):
        c = lax.axis_index("c")
        s = lax.axis_index("s")
        base = (c * _NS + s) * _EPW
        ixs = (ixs0, ixs1)
        ixd = (ixd0, ixd1)
        srow = (srow0, srow1)
        drow = (drow0, drow1)
        fsb = (fsb0, fsb1)
        sis = (si0, si1)
        sgs = (sgs0, sgs1)
        sgd = (sgd0, sgd1)
        sws = (sw0, sw1)

        def start_idx(kk, b):
            off = pl.multiple_of(base + kk * _CH, 8)
            pltpu.async_copy(src_hbm.at[pl.ds(off, _CH)], ixs[b], sis[b])
            pltpu.async_copy(dst_hbm.at[pl.ds(off, _CH)], ixd[b], sis[b])

        def wait_idx(b):
            pltpu.make_async_copy(src_hbm.at[pl.ds(0, _CH)], ixs[b], sis[b]).wait()
            pltpu.make_async_copy(dst_hbm.at[pl.ds(0, _CH)], ixd[b], sis[b]).wait()

        def start_gath(b):
            pltpu.async_copy(lr_hbm.at[ixs[b]], srow[b], sgs[b])
            pltpu.async_copy(lr_hbm.at[ixd[b]], drow[b], sgd[b])

        def wait_gath(b):
            pltpu.make_async_copy(lr_hbm.at[ixs[b]], srow[b], sgs[b]).wait()
            pltpu.make_async_copy(lr_hbm.at[ixd[b]], drow[b], sgd[b]).wait()

        def start_write(kk, b):
            off = pl.multiple_of(base + kk * _CH, 8)
            pltpu.async_copy(fsb[b], out_hbm.at[pl.ds(off, _CH)], sws[b])

        def wait_write(b):
            pltpu.make_async_copy(fsb[b], out_hbm.at[pl.ds(0, _CH)], sws[b]).wait()

        def compute(b):
            sr = srow[b]
            dr = drow[b]
            fb = fsb[b]

            def row(r, carry):
                for j in range(HP // 16):
                    a = sr[r, pl.ds(j * 16, 16)] + dr[r, pl.ds(HP + j * 16, 16)]
                    bb = sr[r, pl.ds(HP + j * 16, 16)] + dr[r, pl.ds(j * 16, 16)]
                    fb[r, pl.ds(j * 16, 16)] = (jnp.maximum(a, 0.0)
                                                + jnp.maximum(bb, 0.0))
                return carry

            lax.fori_loop(0, 1, row, 0)

        start_idx(0, 0)
        wait_idx(0)
        start_gath(0)
        start_idx(1, 1)

        def body(k2, carry):
            for b in range(2):
                kk = k2 * 2 + b

                @pl.when(kk + 1 < _NCH)
                def _():
                    wait_idx(1 - b)
                    start_gath(1 - b)

                wait_gath(b)

                @pl.when(kk >= 2)
                def _():
                    wait_write(b)

                compute(b)
                start_write(kk, b)

                @pl.when(kk + 2 < _NCH)
                def _():
                    start_idx(kk + 2, b)
            return carry

        lax.fori_loop(0, (_NCH - 1) // 2, body, 0)
        wait_gath(0)
        wait_write(0)
        compute(0)
        start_write(_NCH - 1, 0)
        wait_write(1)
        wait_write(0)

    return k(lr, src, dst)


def _node_pre_body(x_ref, wnn_ref, bnn_ref, wl_ref, bl_ref, wr_ref, br_ref,
                   nn0_ref, lr_ref):
    x = x_ref[...]
    nn0_ref[...] = jnp.maximum(
        jnp.dot(x, wnn_ref[...], preferred_element_type=jnp.float32) + bnn_ref[...], 0.0)
    left = jnp.dot(x, wl_ref[...], preferred_element_type=jnp.float32) + bl_ref[...]
    right = jnp.dot(x, wr_ref[...], preferred_element_type=jnp.float32) + br_ref[...]
    lr_ref[...] = jnp.concatenate([left, right], axis=1)


def _node_pre(x, wnn, bnn, wl, bl, wr, br):
    return pl.pallas_call(
        _node_pre_body,
        out_shape=(jax.ShapeDtypeStruct((V, HP), jnp.float32),
                   jax.ShapeDtypeStruct((V, 2 * HP), jnp.float32)),
    )(x, wnn, bnn, wl, bl, wr, br)


def _edge0_body(e_ref, w_ref, b_ref, o_ref):
    o_ref[...] = jnp.maximum(
        jnp.dot(e_ref[...], w_ref[...], preferred_element_type=jnp.float32) + b_ref[...], 0.0)


def _edge0(e, w, b):
    return pl.pallas_call(
        _edge0_body,
        grid=(NEB,),
        in_specs=[pl.BlockSpec((EBLK, DE), lambda i: (i, 0)),
                  pl.BlockSpec((DE, HQ), lambda i: (0, 0)),
                  pl.BlockSpec((1, HQ), lambda i: (0, 0))],
        out_specs=pl.BlockSpec((EBLK, HQ), lambda i: (i, 0)),
        out_shape=jax.ShapeDtypeStruct((E, HQ), jnp.float32),
    )(e, w, b)


def _node_upd_body(nn0_ref, agg_ref, wt_ref, wb_ref, b0_ref, w1_ref, b1_ref, o_ref):
    agg = agg_ref[0, :V, :] + agg_ref[1, :V, :]
    new_x = jnp.maximum(
        jnp.dot(nn0_ref[...], wt_ref[...], preferred_element_type=jnp.float32)
        + jnp.dot(agg, wb_ref[...], preferred_element_type=jnp.float32)
        + b0_ref[...], 0.0)
    o_ref[...] = jnp.maximum(
        jnp.dot(new_x, w1_ref[...], preferred_element_type=jnp.float32) + b1_ref[...], 0.0)


def _node_upd(nn0, agg, wt, wb, b0, w1, b1):
    return pl.pallas_call(
        _node_upd_body,
        out_shape=jax.ShapeDtypeStruct((V, HP), jnp.float32),
    )(nn0, agg, wt, wb, b0, w1, b1)


def _edge1_body(fs_ref, e_ref, wee_ref, bee_ref, wut_ref, wub_ref, bu_ref,
                w1_ref, b1_ref, o_ref):
    third = jnp.maximum(
        jnp.dot(e_ref[...], wee_ref[...], preferred_element_type=jnp.float32) + bee_ref[...], 0.0)
    new_e = jnp.maximum(
        jnp.dot(fs_ref[...], wut_ref[...], preferred_element_type=jnp.float32)
        + jnp.dot(third, wub_ref[...], preferred_element_type=jnp.float32)
        + bu_ref[...], 0.0)
    o_ref[...] = jnp.maximum(
        jnp.dot(new_e, w1_ref[...], preferred_element_type=jnp.float32) + b1_ref[...], 0.0)


def _edge1(fs, e, wee, bee, wut, wub, bu, w1, b1):
    return pl.pallas_call(
        _edge1_body,
        grid=(NEB,),
        in_specs=[pl.BlockSpec((EBLK, HP), lambda i: (i, 0)),
                  pl.BlockSpec((EBLK, DE), lambda i: (i, 0)),
                  pl.BlockSpec((DE, HP), lambda i: (0, 0)),
                  pl.BlockSpec((1, HP), lambda i: (0, 0)),
                  pl.BlockSpec((HP, HP), lambda i: (0, 0)),
                  pl.BlockSpec((HP, HP), lambda i: (0, 0)),
                  pl.BlockSpec((1, HP), lambda i: (0, 0)),
                  pl.BlockSpec((HP, HQ), lambda i: (0, 0)),
                  pl.BlockSpec((1, HQ), lambda i: (0, 0))],
        out_specs=pl.BlockSpec((EBLK, HQ), lambda i: (i, 0)),
        out_shape=jax.ShapeDtypeStruct((E, HQ), jnp.float32),
    )(fs, e, wee, bee, wut, wub, bu, w1, b1)


def _node2_body(nn1_ref, agg_ref, wt_ref, wb_ref, b0_ref, wg_ref, bg_ref,
                t_ref, s_ref):
    agg = agg_ref[0, :V, :] + agg_ref[1, :V, :]
    h2 = jnp.maximum(
        jnp.dot(nn1_ref[...], wt_ref[...], preferred_element_type=jnp.float32)
        + jnp.dot(agg, wb_ref[...], preferred_element_type=jnp.float32)
        + b0_ref[...], 0.0)
    t = jnp.tanh(jnp.dot(h2, wg_ref[...], preferred_element_type=jnp.float32) + bg_ref[...])
    t_ref[...] = t
    s1 = jnp.sum(t, axis=0, keepdims=True)
    s2 = jnp.sum(t * t, axis=0, keepdims=True)
    s_ref[...] = jnp.concatenate([s1, s2, jnp.zeros((6, GF), jnp.float32)], axis=0)


def _node2(nn1, agg, wt, wb, b0, wg, bg):
    return pl.pallas_call(
        _node2_body,
        out_shape=(jax.ShapeDtypeStruct((V, GF), jnp.float32),
                   jax.ShapeDtypeStruct((8, GF), jnp.float32)),
    )(nn1, agg, wt, wb, b0, wg, bg)


def _gather_body(t_ref, gid_ref, mean_ref, istd_ref, gamma_ref, beta_ref, out_ref):
    i = pl.program_id(0)
    h = (t_ref[...] - mean_ref[...]) * istd_ref[...] * gamma_ref[...] + beta_ref[...]
    ms = []
    denom = jnp.zeros_like(h)
    for k in range(K):
        mk = jnp.exp(-0.5 * ((h - _MEANS[k]) / _STDS[k]) ** 2)
        ms.append(mk)
        denom = denom + mk
    inv = 1.0 / denom
    expanded = jnp.concatenate([m * inv for m in ms], axis=1)
    gid = gid_ref[0]
    onehot = (gid == lax.broadcasted_iota(jnp.int32, (G, VBLK), 0)).astype(jnp.float32)

    @pl.when(i == 0)
    def _():
        out_ref[...] = jnp.zeros_like(out_ref)

    out_ref[...] += jnp.dot(onehot, expanded, preferred_element_type=jnp.float32)


def _graph_gather(t, gid3, mean, istd, gamma, beta):
    return pl.pallas_call(
        _gather_body,
        grid=(NVB,),
        in_specs=[pl.BlockSpec((VBLK, GF), lambda i: (i, 0)),
                  pl.BlockSpec((1, 1, VBLK), lambda i: (i, 0, 0)),
                  pl.BlockSpec((1, GF), lambda i: (0, 0)),
                  pl.BlockSpec((1, GF), lambda i: (0, 0)),
                  pl.BlockSpec((1, GF), lambda i: (0, 0)),
                  pl.BlockSpec((1, GF), lambda i: (0, 0))],
        out_specs=pl.BlockSpec((G, K * GF), lambda i: (0, 0)),
        out_shape=jax.ShapeDtypeStruct((G, K * GF), jnp.float32),
    )(t, gid3, mean, istd, gamma, beta)


def _final_body(g_ref, w_ref, b_ref, o_ref):
    o_ref[...] = jnp.tanh(
        jnp.dot(g_ref[...], w_ref[...], preferred_element_type=jnp.float32) + b_ref[...])


def _final(g2, wo2, bo):
    return pl.pallas_call(
        _final_body,
        out_shape=jax.ShapeDtypeStruct((G, GF), jnp.float32),
    )(g2, wo2, bo)


def kernel(node_feats, edge_feats, Wnn0, bnn0, Wen0, ben0, Wun0, bun0, Wl0, bl0,
           Wr0, br0, Wee0, bee0, Wue0, bue0, Wnn1, bnn1, Wen1, ben1, Wun1, bun1,
           Wg, bg, gamma, beta, Wo, bo, edge_index, graph_ids):
    src = edge_index[0]
    dst = edge_index[1]

    wnn0 = _pad2(Wnn0, DN, HP)
    wen0 = _pad2(Wen0, DE, HQ)
    wun0_t = _pad2(Wun0[:H], HP, HP)
    wun0_b = _pad2(Wun0[H:], HQ, HP)
    wl0 = _pad2(Wl0, DN, HP)
    wr0 = _pad2(Wr0, DN, HP)
    wee0 = _pad2(Wee0, DE, HP)
    wue0_t = _pad2(Wue0[:H], HP, HP)
    wue0_b = _pad2(Wue0[H:], HP, HP)
    wnn1 = _pad2(Wnn1, HP, HP)
    wen1 = _pad2(Wen1, HP, HQ)
    wun1_t = _pad2(Wun1[:H], HP, HP)
    wun1_b = _pad2(Wun1[H:], HQ, HP)
    wg = _pad2(Wg, HP, GF)
    wo2 = Wo.reshape(GF, K, GF).transpose(1, 0, 2).reshape(K * GF, GF)

    nn0, lr = _node_pre(node_feats, wnn0, _pad_b(bnn0, HP), wl0, _pad_b(bl0, HP),
                        wr0, _pad_b(br0, HP))
    e2n0 = _edge0(edge_feats, wen0, _pad_b(ben0, HQ))

    agg0 = _sc_segment_sum(e2n0, dst)
    fs = _sc_edge_gather(lr, src, dst)

    nn1 = _node_upd(nn0, agg0, wun0_t, wun0_b, _pad_b(bun0, HP), wnn1, _pad_b(bnn1, HP))
    e2n1 = _edge1(fs, edge_feats, wee0, _pad_b(bee0, HP), wue0_t, wue0_b,
                  _pad_b(bue0, HP), wen1, _pad_b(ben1, HQ))
    agg1 = _sc_segment_sum(e2n1, dst)

    t, s = _node2(nn1, agg1, wun1_t, wun1_b, _pad_b(bun1, HP), wg, bg.reshape(1, GF))
    mean = s[0:1, :] / V
    var = s[1:2, :] / V - mean * mean
    istd = 1.0 / jnp.sqrt(var + 1e-5)

    g2 = _graph_gather(t, graph_ids.reshape(NVB, 1, VBLK), mean, istd,
                       gamma.reshape(1, GF), beta.reshape(1, GF))
    return _final(g2, wo2, bo.reshape(1, GF))


# speedup vs baseline: 1.0790x; 1.0790x over previous
import functools

import jax
import jax.numpy as jnp
from jax import lax
from jax.experimental import pallas as pl
from jax.experimental.pallas import tpu as pltpu
from jax.experimental.pallas import tpu_sc as plsc

V = 10000
E = 320000
DN = 128
DE = 16
H = 50
HP = 64
HQ = 128
GF = 128
G = 128
K = 11

_MEANS = (-1.645, -1.080, -0.739, -0.468, -0.228, 0.0, 0.228, 0.468, 0.739, 1.080, 1.645)
_STDS = (0.283, 0.170, 0.134, 0.118, 0.114, 0.114, 0.114, 0.118, 0.134, 0.170, 0.283)

EBLK = 8000
NEB = E // EBLK
VBLK = 1000
NVB = V // VBLK


def _pad2(W, rows, cols):
    r, c = W.shape
    return jnp.pad(W, ((0, rows - r), (0, cols - c)))


def _pad_b(b, cols):
    return jnp.pad(b, (0, cols - b.shape[0])).reshape(1, cols)


_NC = 2
_NS = 16
_CH = 80
_EPW = E // (_NC * _NS)
_NCH = _EPW // _CH
_VPAD = 10240
_VPS = _VPAD // _NS
_VOS = V // _NS
_CHG = 40
_NCHG = _EPW // _CHG


def _sc_segment_sum(data, idx):
    mesh = plsc.VectorSubcoreMesh(core_axis_name="c", subcore_axis_name="s")

    @functools.partial(
        pl.kernel, mesh=mesh,
        out_type=jax.ShapeDtypeStruct((_NC, _VPAD, HQ), jnp.float32),
        scratch_types=[
            pltpu.VMEM_SHARED((_VPAD, HQ), jnp.float32),
            pltpu.VMEM((128, HQ), jnp.float32),
            pltpu.VMEM((_CH,), jnp.int32), pltpu.VMEM((_CH,), jnp.int32),
            pltpu.VMEM((_CH, HQ), jnp.float32), pltpu.VMEM((_CH, HQ), jnp.float32),
            pltpu.SemaphoreType.DMA, pltpu.SemaphoreType.DMA,
            pltpu.SemaphoreType.DMA, pltpu.SemaphoreType.DMA,
        ],
    )
    def k(data_hbm, idx_hbm, out_hbm, acc_sh, zbuf, idx0, idx1, rows0, rows1,
          si0, si1, sr0, sr1):
        c = lax.axis_index("c")
        s = lax.axis_index("s")
        z16 = jnp.zeros((16,), jnp.float32)

        def zrow(i, carry):
            for j in range(HQ // 16):
                zbuf[i, pl.ds(j * 16, 16)] = z16
            return carry

        lax.fori_loop(0, 128, zrow, 0)
        for m in range(_VPS // 128):
            pltpu.sync_copy(zbuf, acc_sh.at[pl.ds(s * _VPS + m * 128, 128)])
        plsc.subcore_barrier()

        base = (c * _NS + s) * _EPW
        idxb = (idx0, idx1)
        rowsb = (rows0, rows1)
        sib = (si0, si1)
        srb = (sr0, sr1)

        def start(kk, b):
            off = pl.multiple_of(base + kk * _CH, 8)
            pltpu.async_copy(idx_hbm.at[pl.ds(off, _CH)], idxb[b], sib[b])
            pltpu.async_copy(data_hbm.at[pl.ds(off, _CH)], rowsb[b], srb[b])

        def wait(b):
            pltpu.make_async_copy(idx_hbm.at[pl.ds(0, _CH)], idxb[b], sib[b]).wait()
            pltpu.make_async_copy(data_hbm.at[pl.ds(0, _CH)], rowsb[b], srb[b]).wait()

        def scat(b):
            pltpu.sync_copy(rowsb[b], acc_sh.at[idxb[b]], add=True)

        start(0, 0)
        start(1, 1)

        def body(k2, carry):
            for b in range(2):
                kk = k2 * 2 + b
                wait(b)
                scat(b)

                @pl.when(kk + 2 < _NCH)
                def _():
                    start(kk + 2, b)
            return carry

        lax.fori_loop(0, (_NCH - 1) // 2, body, 0)
        wait(0)
        scat(0)
        plsc.subcore_barrier()
        pltpu.sync_copy(acc_sh.at[pl.ds(s * _VPS, _VPS)],
                        out_hbm.at[c, pl.ds(s * _VPS, _VPS)])

    return k(data, idx)


def _sc_edge_gather(lr, src, dst):
    mesh = plsc.VectorSubcoreMesh(core_axis_name="c", subcore_axis_name="s")

    @functools.partial(
        pl.kernel, mesh=mesh,
        out_type=jax.ShapeDtypeStruct((E, HP), jnp.float32),
        scratch_types=[
            pltpu.VMEM_SHARED((V, 2 * HP), jnp.float32),
            pltpu.VMEM((_CHG,), jnp.int32), pltpu.VMEM((_CHG,), jnp.int32),
            pltpu.VMEM((_CHG,), jnp.int32), pltpu.VMEM((_CHG,), jnp.int32),
            pltpu.VMEM((_CHG, 2 * HP), jnp.float32), pltpu.VMEM((_CHG, 2 * HP), jnp.float32),
            pltpu.VMEM((_CHG, 2 * HP), jnp.float32), pltpu.VMEM((_CHG, 2 * HP), jnp.float32),
            pltpu.VMEM((_CHG, HP), jnp.float32), pltpu.VMEM((_CHG, HP), jnp.float32),
            pltpu.SemaphoreType.DMA, pltpu.SemaphoreType.DMA,
            pltpu.SemaphoreType.DMA, pltpu.SemaphoreType.DMA,
            pltpu.SemaphoreType.DMA, pltpu.SemaphoreType.DMA,
            pltpu.SemaphoreType.DMA, pltpu.SemaphoreType.DMA,
        ],
    )
    def k(lr_hbm, src_hbm, dst_hbm, out_hbm, lr_sh, ixs0, ixs1, ixd0, ixd1,
          srow0, srow1, drow0, drow1, fsb0, fsb1,
          si0, si1, sgs0, sgs1, sgd0, sgd1, sw0, sw1):
        c = lax.axis_index("c")
        s = lax.axis_index("s")
        base = (c * _NS + s) * _EPW

        @pl.when(s < _NS - 1)
        def _():
            pltpu.sync_copy(lr_hbm.at[pl.ds(s * _VPS, _VPS)],
                            lr_sh.at[pl.ds(s * _VPS, _VPS)])

        @pl.when(s == _NS - 1)
        def _():
            pltpu.sync_copy(lr_hbm.at[pl.ds((_NS - 1) * _VPS, V - (_NS - 1) * _VPS)],
                            lr_sh.at[pl.ds((_NS - 1) * _VPS, V - (_NS - 1) * _VPS)])

        plsc.subcore_barrier()

        ixs = (ixs0, ixs1)
        ixd = (ixd0, ixd1)
        srow = (srow0, srow1)
        drow = (drow0, drow1)
        fsb = (fsb0, fsb1)
        sis = (si0, si1)
        sgs = (sgs0, sgs1)
        sgd = (sgd0, sgd1)
        sws = (sw0, sw1)

        def start_idx(kk, b):
            off = pl.multiple_of(base + kk * _CHG, 8)
            pltpu.async_copy(src_hbm.at[pl.ds(off, _CHG)], ixs[b], sis[b])
            pltpu.async_copy(dst_hbm.at[pl.ds(off, _CHG)], ixd[b], sis[b])

        def wait_idx(b):
            pltpu.make_async_copy(src_hbm.at[pl.ds(0, _CHG)], ixs[b], sis[b]).wait()
            pltpu.make_async_copy(dst_hbm.at[pl.ds(0, _CHG)], ixd[b], sis[b]).wait()

        def start_gath(b):
            pltpu.async_copy(lr_sh.at[ixs[b]], srow[b], sgs[b])
            pltpu.async_copy(lr_sh.at[ixd[b]], drow[b], sgd[b])

        def wait_gath(b):
            pltpu.make_async_copy(lr_sh.at[ixs[b]], srow[b], sgs[b]).wait()
            pltpu.make_async_copy(lr_sh.at[ixd[b]], drow[b], sgd[b]).wait()

        def start_write(kk, b):
            off = pl.multiple_of(base + kk * _CHG, 8)
            pltpu.async_copy(fsb[b], out_hbm.at[pl.ds(off, _CHG)], sws[b])

        def wait_write(b):
            pltpu.make_async_copy(fsb[b], out_hbm.at[pl.ds(0, _CHG)], sws[b]).wait()

        def compute(b):
            sr = srow[b]
            dr = drow[b]
            fb = fsb[b]

            def row(r, carry):
                for j in range(HP // 16):
                    a = sr[r, pl.ds(j * 16, 16)] + dr[r, pl.ds(HP + j * 16, 16)]
                    bb = sr[r, pl.ds(HP + j * 16, 16)] + dr[r, pl.ds(j * 16, 16)]
                    fb[r, pl.ds(j * 16, 16)] = (jnp.maximum(a, 0.0)
                                                + jnp.maximum(bb, 0.0))
                return carry

            lax.fori_loop(0, _CHG, row, 0)

        start_idx(0, 0)
        wait_idx(0)
        start_gath(0)
        start_idx(1, 1)

        def body(k2, carry):
            for b in range(2):
                kk = k2 * 2 + b

                @pl.when(kk + 1 < _NCHG)
                def _():
                    wait_idx(1 - b)
                    start_gath(1 - b)

                wait_gath(b)

                @pl.when(kk >= 2)
                def _():
                    wait_write(b)

                compute(b)
                start_write(kk, b)

                @pl.when(kk + 2 < _NCHG)
                def _():
                    start_idx(kk + 2, b)
            return carry

        lax.fori_loop(0, (_NCHG - 2) // 2, body, 0)
        wait_idx(1)
        start_gath(1)
        wait_gath(0)
        wait_write(0)
        compute(0)
        start_write(_NCHG - 2, 0)
        wait_gath(1)
        wait_write(1)
        compute(1)
        start_write(_NCHG - 1, 1)
        wait_write(0)
        wait_write(1)

    return k(lr, src, dst)


def _node_pre_body(x_ref, wnn_ref, bnn_ref, wl_ref, bl_ref, wr_ref, br_ref,
                   nn0_ref, lr_ref):
    x = x_ref[...]
    nn0_ref[...] = jnp.maximum(
        jnp.dot(x, wnn_ref[...], preferred_element_type=jnp.float32) + bnn_ref[...], 0.0)
    left = jnp.dot(x, wl_ref[...], preferred_element_type=jnp.float32) + bl_ref[...]
    right = jnp.dot(x, wr_ref[...], preferred_element_type=jnp.float32) + br_ref[...]
    lr_ref[...] = jnp.concatenate([left, right], axis=1)


def _node_pre(x, wnn, bnn, wl, bl, wr, br):
    return pl.pallas_call(
        _node_pre_body,
        out_shape=(jax.ShapeDtypeStruct((V, HP), jnp.float32),
                   jax.ShapeDtypeStruct((V, 2 * HP), jnp.float32)),
    )(x, wnn, bnn, wl, bl, wr, br)


def _edge0_body(e_ref, w_ref, b_ref, o_ref):
    o_ref[...] = jnp.maximum(
        jnp.dot(e_ref[...], w_ref[...], preferred_element_type=jnp.float32) + b_ref[...], 0.0)


def _edge0(e, w, b):
    return pl.pallas_call(
        _edge0_body,
        grid=(NEB,),
        in_specs=[pl.BlockSpec((EBLK, DE), lambda i: (i, 0)),
                  pl.BlockSpec((DE, HQ), lambda i: (0, 0)),
                  pl.BlockSpec((1, HQ), lambda i: (0, 0))],
        out_specs=pl.BlockSpec((EBLK, HQ), lambda i: (i, 0)),
        out_shape=jax.ShapeDtypeStruct((E, HQ), jnp.float32),
    )(e, w, b)


def _node_upd_body(nn0_ref, agg_ref, wt_ref, wb_ref, b0_ref, w1_ref, b1_ref, o_ref):
    agg = agg_ref[0, :V, :] + agg_ref[1, :V, :]
    new_x = jnp.maximum(
        jnp.dot(nn0_ref[...], wt_ref[...], preferred_element_type=jnp.float32)
        + jnp.dot(agg, wb_ref[...], preferred_element_type=jnp.float32)
        + b0_ref[...], 0.0)
    o_ref[...] = jnp.maximum(
        jnp.dot(new_x, w1_ref[...], preferred_element_type=jnp.float32) + b1_ref[...], 0.0)


def _node_upd(nn0, agg, wt, wb, b0, w1, b1):
    return pl.pallas_call(
        _node_upd_body,
        out_shape=jax.ShapeDtypeStruct((V, HP), jnp.float32),
    )(nn0, agg, wt, wb, b0, w1, b1)


def _edge1_body(fs_ref, e_ref, wee_ref, bee_ref, wut_ref, wub_ref, bu_ref,
                w1_ref, b1_ref, o_ref):
    third = jnp.maximum(
        jnp.dot(e_ref[...], wee_ref[...], preferred_element_type=jnp.float32) + bee_ref[...], 0.0)
    new_e = jnp.maximum(
        jnp.dot(fs_ref[...], wut_ref[...], preferred_element_type=jnp.float32)
        + jnp.dot(third, wub_ref[...], preferred_element_type=jnp.float32)
        + bu_ref[...], 0.0)
    o_ref[...] = jnp.maximum(
        jnp.dot(new_e, w1_ref[...], preferred_element_type=jnp.float32) + b1_ref[...], 0.0)


def _edge1(fs, e, wee, bee, wut, wub, bu, w1, b1):
    return pl.pallas_call(
        _edge1_body,
        grid=(NEB,),
        in_specs=[pl.BlockSpec((EBLK, HP), lambda i: (i, 0)),
                  pl.BlockSpec((EBLK, DE), lambda i: (i, 0)),
                  pl.BlockSpec((DE, HP), lambda i: (0, 0)),
                  pl.BlockSpec((1, HP), lambda i: (0, 0)),
                  pl.BlockSpec((HP, HP), lambda i: (0, 0)),
                  pl.BlockSpec((HP, HP), lambda i: (0, 0)),
                  pl.BlockSpec((1, HP), lambda i: (0, 0)),
                  pl.BlockSpec((HP, HQ), lambda i: (0, 0)),
                  pl.BlockSpec((1, HQ), lambda i: (0, 0))],
        out_specs=pl.BlockSpec((EBLK, HQ), lambda i: (i, 0)),
        out_shape=jax.ShapeDtypeStruct((E, HQ), jnp.float32),
    )(fs, e, wee, bee, wut, wub, bu, w1, b1)


def _node2_body(nn1_ref, agg_ref, wt_ref, wb_ref, b0_ref, wg_ref, bg_ref,
                t_ref, s_ref):
    agg = agg_ref[0, :V, :] + agg_ref[1, :V, :]
    h2 = jnp.maximum(
        jnp.dot(nn1_ref[...], wt_ref[...], preferred_element_type=jnp.float32)
        + jnp.dot(agg, wb_ref[...], preferred_element_type=jnp.float32)
        + b0_ref[...], 0.0)
    t = jnp.tanh(jnp.dot(h2, wg_ref[...], preferred_element_type=jnp.float32) + bg_ref[...])
    t_ref[...] = t
    s1 = jnp.sum(t, axis=0, keepdims=True)
    s2 = jnp.sum(t * t, axis=0, keepdims=True)
    s_ref[...] = jnp.concatenate([s1, s2, jnp.zeros((6, GF), jnp.float32)], axis=0)


def _node2(nn1, agg, wt, wb, b0, wg, bg):
    return pl.pallas_call(
        _node2_body,
        out_shape=(jax.ShapeDtypeStruct((V, GF), jnp.float32),
                   jax.ShapeDtypeStruct((8, GF), jnp.float32)),
    )(nn1, agg, wt, wb, b0, wg, bg)


def _gather_body(t_ref, gid_ref, mean_ref, istd_ref, gamma_ref, beta_ref, out_ref):
    i = pl.program_id(0)
    h = (t_ref[...] - mean_ref[...]) * istd_ref[...] * gamma_ref[...] + beta_ref[...]
    ms = []
    denom = jnp.zeros_like(h)
    for k in range(K):
        mk = jnp.exp(-0.5 * ((h - _MEANS[k]) / _STDS[k]) ** 2)
        ms.append(mk)
        denom = denom + mk
    inv = 1.0 / denom
    expanded = jnp.concatenate([m * inv for m in ms], axis=1)
    gid = gid_ref[0]
    onehot = (gid == lax.broadcasted_iota(jnp.int32, (G, VBLK), 0)).astype(jnp.float32)

    @pl.when(i == 0)
    def _():
        out_ref[...] = jnp.zeros_like(out_ref)

    out_ref[...] += jnp.dot(onehot, expanded, preferred_element_type=jnp.float32)


def _graph_gather(t, gid3, mean, istd, gamma, beta):
    return pl.pallas_call(
        _gather_body,
        grid=(NVB,),
        in_specs=[pl.BlockSpec((VBLK, GF), lambda i: (i, 0)),
                  pl.BlockSpec((1, 1, VBLK), lambda i: (i, 0, 0)),
                  pl.BlockSpec((1, GF), lambda i: (0, 0)),
                  pl.BlockSpec((1, GF), lambda i: (0, 0)),
                  pl.BlockSpec((1, GF), lambda i: (0, 0)),
                  pl.BlockSpec((1, GF), lambda i: (0, 0))],
        out_specs=pl.BlockSpec((G, K * GF), lambda i: (0, 0)),
        out_shape=jax.ShapeDtypeStruct((G, K * GF), jnp.float32),
    )(t, gid3, mean, istd, gamma, beta)


def _final_body(g_ref, w_ref, b_ref, o_ref):
    o_ref[...] = jnp.tanh(
        jnp.dot(g_ref[...], w_ref[...], preferred_element_type=jnp.float32) + b_ref[...])


def _final(g2, wo2, bo):
    return pl.pallas_call(
        _final_body,
        out_shape=jax.ShapeDtypeStruct((G, GF), jnp.float32),
    )(g2, wo2, bo)


def kernel(node_feats, edge_feats, Wnn0, bnn0, Wen0, ben0, Wun0, bun0, Wl0, bl0,
           Wr0, br0, Wee0, bee0, Wue0, bue0, Wnn1, bnn1, Wen1, ben1, Wun1, bun1,
           Wg, bg, gamma, beta, Wo, bo, edge_index, graph_ids):
    src = edge_index[0]
    dst = edge_index[1]

    wnn0 = _pad2(Wnn0, DN, HP)
    wen0 = _pad2(Wen0, DE, HQ)
    wun0_t = _pad2(Wun0[:H], HP, HP)
    wun0_b = _pad2(Wun0[H:], HQ, HP)
    wl0 = _pad2(Wl0, DN, HP)
    wr0 = _pad2(Wr0, DN, HP)
    wee0 = _pad2(Wee0, DE, HP)
    wue0_t = _pad2(Wue0[:H], HP, HP)
    wue0_b = _pad2(Wue0[H:], HP, HP)
    wnn1 = _pad2(Wnn1, HP, HP)
    wen1 = _pad2(Wen1, HP, HQ)
    wun1_t = _pad2(Wun1[:H], HP, HP)
    wun1_b = _pad2(Wun1[H:], HQ, HP)
    wg = _pad2(Wg, HP, GF)
    wo2 = Wo.reshape(GF, K, GF).transpose(1, 0, 2).reshape(K * GF, GF)

    nn0, lr = _node_pre(node_feats, wnn0, _pad_b(bnn0, HP), wl0, _pad_b(bl0, HP),
                        wr0, _pad_b(br0, HP))
    e2n0 = _edge0(edge_feats, wen0, _pad_b(ben0, HQ))

    agg0 = _sc_segment_sum(e2n0, dst)
    fs = _sc_edge_gather(lr, src, dst)

    nn1 = _node_upd(nn0, agg0, wun0_t, wun0_b, _pad_b(bun0, HP), wnn1, _pad_b(bnn1, HP))
    e2n1 = _edge1(fs, edge_feats, wee0, _pad_b(bee0, HP), wue0_t, wue0_b,
                  _pad_b(bue0, HP), wen1, _pad_b(ben1, HQ))
    agg1 = _sc_segment_sum(e2n1, dst)

    t, s = _node2(nn1, agg1, wun1_t, wun1_b, _pad_b(bun1, HP), wg, bg.reshape(1, GF))
    mean = s[0:1, :] / V
    var = s[1:2, :] / V - mean * mean
    istd = 1.0 / jnp.sqrt(var + 1e-5)

    g2 = _graph_gather(t, graph_ids.reshape(NVB, 1, VBLK), mean, istd,
                       gamma.reshape(1, GF), beta.reshape(1, GF))
    return _final(g2, wo2, bo.reshape(1, GF))
